# Initial kernel scaffold; baseline (speedup 1.0000x reference)
import jax
import jax.numpy as jnp
from jax.experimental import pallas as pl


def kernel(nodes, neigh_idx, features, weight):
    raise NotImplementedError("write your pallas kernel here")


# speedup vs baseline: 6.3553x; 6.3553x over previous
import functools

import jax
import jax.numpy as jnp
from jax import lax
from jax.experimental import pallas as pl
from jax.experimental.pallas import tpu as pltpu
from jax.experimental.pallas import tpu_sc as plsc

B = 16384
S = 25
F = 128
E = 128
L = 16
NC, NS = 2, 16
NW = NC * NS
BPW = B // NW
CH = 4
NCHUNK = BPW // CH
SG = B // (NW * 128)

def _accum_chunk(nrows, obuf):
    for d in range(CH):
        r0 = d * S
        for j in range(F // L):
            sl = pl.ds(j * L, L)
            acc = nrows[r0, sl]
            for s in range(1, S):
                acc = acc + nrows[r0 + s, sl]
            obuf[d, sl] = acc


@functools.cache
def _build_sc_gather():
  mesh = plsc.VectorSubcoreMesh(core_axis_name="c", subcore_axis_name="s")

  @functools.partial(
    pl.kernel,
    out_type=[
        jax.ShapeDtypeStruct((B, F), jnp.float32),
        jax.ShapeDtypeStruct((B, F), jnp.float32),
    ],
    mesh=mesh,
    scratch_types=[
        pltpu.VMEM((SG, 128), jnp.int32),
        pltpu.VMEM((NCHUNK, CH * S), jnp.int32),
        pltpu.VMEM((2, 128, F), jnp.float32),
        pltpu.VMEM((CH * S, F), jnp.float32),
        pltpu.VMEM((CH * S, F), jnp.float32),
        pltpu.VMEM((CH, F), jnp.float32),
        pltpu.VMEM((CH, F), jnp.float32),
        pltpu.SemaphoreType.DMA,
        pltpu.SemaphoreType.DMA,
        pltpu.SemaphoreType.DMA,
        pltpu.SemaphoreType.DMA,
    ],
)
  def _sc_gather(nodes2, neigh2, feat, self_out, neigh_out,
                 nidx, eidx, srows, nrows0, nrows1, obuf0, obuf1,
                 gsem0, gsem1, wsem0, wsem1):
      wid = lax.axis_index("s") * NC + lax.axis_index("c")
      obase = wid * BPW

      pltpu.sync_copy(nodes2.at[pl.ds(wid * SG, SG)], nidx)
      pltpu.sync_copy(neigh2.at[pl.ds(wid * NCHUNK, NCHUNK)], eidx)

      nrows = (nrows0, nrows1)
      obufs = (obuf0, obuf1)
      gsems = (gsem0, gsem1)
      wsems = (wsem0, wsem1)

      pltpu.make_async_copy(feat.at[nidx.at[0]], srows.at[0], gsems[0]).start()
      pltpu.make_async_copy(feat.at[nidx.at[1]], srows.at[1], gsems[1]).start()
      for g in range(SG):
          p = g % 2
          pltpu.make_async_copy(feat.at[nidx.at[g]], srows.at[p], gsems[p]).wait()
          out_sl = self_out.at[pl.ds(obase + g * 128, 128)]
          pltpu.make_async_copy(srows.at[p], out_sl, wsems[p]).start()
          if g + 2 < SG:
              pltpu.make_async_copy(srows.at[p], out_sl, wsems[p]).wait()
              pltpu.make_async_copy(feat.at[nidx.at[g + 2]], srows.at[p], gsems[p]).start()
      for g in range(SG - 2, SG):
          p = g % 2
          out_sl = self_out.at[pl.ds(obase + g * 128, 128)]
          pltpu.make_async_copy(srows.at[p], out_sl, wsems[p]).wait()

      pltpu.make_async_copy(feat.at[eidx.at[0]], nrows[0], gsems[0]).start()
      pltpu.make_async_copy(feat.at[eidx.at[1]], nrows[1], gsems[1]).start()

      def body(c2, carry):
          for k in range(2):
              c = c2 * 2 + k

              @pl.when(c >= 2)
              def _wait_write():
                  dst = neigh_out.at[pl.ds(obase + (c - 2) * CH, CH)]
                  pltpu.make_async_copy(obufs[k], dst, wsems[k]).wait()

              pltpu.make_async_copy(feat.at[eidx.at[c]], nrows[k], gsems[k]).wait()
              _accum_chunk(nrows[k], obufs[k])

              @pl.when(c + 2 < NCHUNK)
              def _next_gather():
                  pltpu.make_async_copy(
                      feat.at[eidx.at[c + 2]], nrows[k], gsems[k]).start()

              dst = neigh_out.at[pl.ds(obase + c * CH, CH)]
              pltpu.make_async_copy(obufs[k], dst, wsems[k]).start()
          return carry

      lax.fori_loop(0, NCHUNK // 2, body, 0)

      for c in (NCHUNK - 2, NCHUNK - 1):
          k = c % 2
          dst = neigh_out.at[pl.ds(obase + c * CH, CH)]
          pltpu.make_async_copy(obufs[k], dst, wsems[k]).wait()

  return _sc_gather


def _tc_body(w_ref, s_ref, n_ref, o_ref):
    w1 = w_ref[:, :F]
    w2 = w_ref[:, F:]
    dn = (((1,), (1,)), ((), ()))
    acc = lax.dot_general(w1, s_ref[...], dn, preferred_element_type=jnp.float32)
    acc = acc + lax.dot_general(w2, n_ref[...], dn, preferred_element_type=jnp.float32)
    o_ref[...] = jnp.maximum(acc, 0.0)


_BLK = 2048


@jax.jit
def _tc_matmul(w, self_feats, neigh_sums):
    return pl.pallas_call(
        _tc_body,
        out_shape=jax.ShapeDtypeStruct((E, B), jnp.float32),
        grid=(B // _BLK,),
        in_specs=[
            pl.BlockSpec((E, 2 * F), lambda i: (0, 0)),
            pl.BlockSpec((_BLK, F), lambda i: (i, 0)),
            pl.BlockSpec((_BLK, F), lambda i: (i, 0)),
        ],
        out_specs=pl.BlockSpec((E, _BLK), lambda i: (0, i)),
    )(w, self_feats, neigh_sums)


def kernel(nodes, neigh_idx, features, weight):
    nodes2 = nodes.reshape(B // 128, 128).astype(jnp.int32)
    neigh2 = neigh_idx.reshape(B * S // (CH * S), CH * S).astype(jnp.int32)
    self_feats, neigh_sums = _build_sc_gather()(nodes2, neigh2, features)
    wscaled = jnp.concatenate([weight[:, :F], weight[:, F:] * (1.0 / S)], axis=1)
    return _tc_matmul(wscaled, self_feats, neigh_sums)
